# baseline (speedup 1.0000x reference)
import functools
import math

import jax
import jax.numpy as jnp
from jax import lax
from jax.experimental import pallas as pl
from jax.experimental.pallas import tpu as pltpu
from jax.experimental.pallas import tpu_sc as plsc

D_MODEL = 64
SCALE = math.sqrt(D_MODEL)

_INFO = plsc.get_sparse_core_info()
NW = _INFO.num_cores * _INFO.num_subcores

R1 = 320
SEQ = 200


@functools.lru_cache(maxsize=None)
def _build_repack(V):
    n_chunks = V // R1
    per_tile = -(-n_chunks // NW)
    n_pairs = ((per_tile + 1) // 2) * 2

    mesh = plsc.VectorSubcoreMesh(core_axis_name="c", subcore_axis_name="s")

    @functools.partial(
        pl.kernel,
        mesh=mesh,
        out_type=jax.ShapeDtypeStruct((V // 2, 2 * D_MODEL), jnp.float32),
        scratch_types=[
            pltpu.VMEM((R1, D_MODEL), jnp.float32),
            pltpu.VMEM((2, R1 // 2, 2 * D_MODEL), jnp.float32),
            pltpu.SemaphoreType.DMA,
            pltpu.SemaphoreType.DMA,
        ],
    )
    def repack_kernel(tbl_hbm, cmp_hbm, a_v, b_v, *osem):
        cid = lax.axis_index("c")
        sid = lax.axis_index("s")
        wid = sid * _INFO.num_cores + cid

        def pair_body(g, carry):
            for b in range(2):
                t = g * 2 + b
                c = wid + t * NW

                @pl.when(c < n_chunks)
                def _():
                    r0 = pl.multiple_of(c * R1, R1)
                    p0 = pl.multiple_of(c * (R1 // 2), R1 // 2)
                    @pl.when(t >= 2)
                    def _():
                        pltpu.make_async_copy(
                            b_v.at[b],
                            cmp_hbm.at[pl.ds(0, R1 // 2)],
                            osem[b],
                        ).wait()

                    pltpu.sync_copy(tbl_hbm.at[pl.ds(r0, R1)], a_v)

                    def rows(q, carry2):
                        for half in range(2):
                            for d in range(D_MODEL // 16):
                                b_v[b, q,
                                    pl.ds(half * D_MODEL + d * 16, 16)] = (
                                    a_v[2 * q + half, pl.ds(d * 16, 16)]
                                    * SCALE)
                        return carry2

                    lax.fori_loop(0, R1 // 2, rows, 0, unroll=4)

                    pltpu.async_copy(
                        b_v.at[b],
                        cmp_hbm.at[pl.ds(p0, R1 // 2)],
                        osem[b],
                    )
            return carry

        lax.fori_loop(0, n_pairs // 2, pair_body, 0)
        my_chunks = (n_chunks - 1 - wid) // NW + 1
        for b in range(2):
            @pl.when(my_chunks > b)
            def _():
                pltpu.make_async_copy(
                    b_v.at[b],
                    cmp_hbm.at[pl.ds(0, R1 // 2)],
                    osem[b],
                ).wait()

    return repack_kernel


@functools.lru_cache(maxsize=None)
def _build_gather(NB, V):
    b_per_w = NB // NW
    G0, G1 = 104, SEQ - 104

    mesh = plsc.VectorSubcoreMesh(core_axis_name="c", subcore_axis_name="s")

    @functools.partial(
        pl.kernel,
        mesh=mesh,
        compiler_params=pltpu.CompilerParams(use_tc_tiling_on_sc=False),
        out_type=jax.ShapeDtypeStruct((NB * SEQ, 2 * D_MODEL), jnp.float32),
        scratch_types=[
            pltpu.VMEM((2, 208), jnp.int32),
            pltpu.VMEM((2, 208), jnp.int32),
            pltpu.VMEM((2, SEQ, 2 * D_MODEL), jnp.float32),
            pltpu.VMEM((2, SEQ, 2 * D_MODEL), jnp.float32),
            pltpu.SemaphoreType.DMA,
            pltpu.SemaphoreType.DMA,
            pltpu.SemaphoreType.DMA,
            pltpu.SemaphoreType.DMA,
        ],
    )
    def gather_kernel(idx_hbm, cmp_hbm, out_hbm, idx_v, jdx_v, ga_v, sc_v,
                      *sems):
        gsem = sems[:2]
        osem = sems[2:]
        cid = lax.axis_index("c")
        sid = lax.axis_index("s")
        wid = sid * _INFO.num_cores + cid
        bat_base = wid * b_per_w

        def stage_and_fire(ci, s):
            pltpu.sync_copy(
                idx_hbm.at[pl.ds((bat_base + ci) * SEQ, SEQ)],
                idx_v.at[s, pl.ds(0, SEQ)])
            for v in range(13):
                sl = pl.ds(v * 16, 16)
                jdx_v[s, sl] = idx_v[s, sl] >> 1
            for o, g in ((0, G0), (G0, G1)):
                pltpu.async_copy(
                    cmp_hbm.at[jdx_v.at[s, pl.ds(o, g)]],
                    ga_v.at[s, pl.ds(o, g)],
                    gsem[s],
                )

        def wait_gather(s):
            pltpu.make_async_copy(
                cmp_hbm.at[pl.ds(0, SEQ)], ga_v.at[s], gsem[s]).wait()

        def out_slice(ci):
            r0 = pl.multiple_of((bat_base + ci) * SEQ, SEQ)
            return out_hbm.at[pl.ds(r0, SEQ)]

        def wait_out(ci, s):
            pltpu.make_async_copy(sc_v.at[s], out_slice(ci), osem[s]).wait()

        stage_and_fire(0, 0)
        stage_and_fire(1, 1)

        def chunk_pair(g, carry):
            for b in range(2):
                ci = g * 2 + b
                wait_gather(b)

                @pl.when(ci >= 2)
                def _():
                    wait_out(ci - 2, b)

                def sel_rows(grp, nk):
                    iv = idx_v[b, pl.ds(grp * 16, 16)]
                    for k in range(nk):
                        r = grp * 16 + k
                        off = (iv[k] & 1) * D_MODEL
                        for d in range(D_MODEL // 16):
                            sc_v[b, r, pl.ds(d * 16, 16)] = (
                                ga_v[b, r, pl.ds(off + d * 16, 16)])

                def sel_group(grp, carry2):
                    sel_rows(grp, 16)
                    return carry2

                lax.fori_loop(0, SEQ // 16, sel_group, 0)
                sel_rows(SEQ // 16, SEQ % 16)

                pltpu.async_copy(sc_v.at[b], out_slice(ci), osem[b])

                @pl.when(ci + 2 < b_per_w)
                def _():
                    stage_and_fire(ci + 2, b)
            return carry

        lax.fori_loop(0, b_per_w // 2, chunk_pair, 0)
        wait_out(b_per_w - 2, 0)
        wait_out(b_per_w - 1, 1)

    return gather_kernel


def kernel(x, pretrained_vector):
    NB, S = x.shape
    V = pretrained_vector.shape[0]
    idx = x.reshape(NB * S).astype(jnp.int32)
    cmp = _build_repack(V)(pretrained_vector)
    out = _build_gather(NB, V)(idx, cmp)
    return out.reshape(NB, S, 2 * D_MODEL)[:, :, :D_MODEL]


# speedup vs baseline: 1.4773x; 1.4773x over previous
import functools
import math

import jax
import jax.numpy as jnp
from jax import lax
from jax.experimental import pallas as pl
from jax.experimental.pallas import tpu as pltpu
from jax.experimental.pallas import tpu_sc as plsc

D_MODEL = 64
SCALE = math.sqrt(D_MODEL)

G = 128
K = 2
C = G * K
NBUF = 4


@functools.lru_cache(maxsize=None)
def _build(B):
    info = plsc.get_sparse_core_info()
    NW = info.num_cores * info.num_subcores
    assert B % (NW * C * NBUF) == 0
    b_per_w = B // NW
    n_chunks = b_per_w // C
    n_groups = n_chunks // NBUF
    g_rows_per_w = b_per_w // G

    mesh = plsc.VectorSubcoreMesh(core_axis_name="c", subcore_axis_name="s")

    @functools.partial(
        pl.kernel,
        mesh=mesh,
        compiler_params=pltpu.CompilerParams(use_tc_tiling_on_sc=False),
        out_type=jax.ShapeDtypeStruct((B, D_MODEL), jnp.float32),
        scratch_types=[
            pltpu.VMEM((g_rows_per_w, G), jnp.int32),
            pltpu.VMEM((NBUF, C, D_MODEL), jnp.float32),
            pltpu.SemaphoreType.DMA,
            pltpu.SemaphoreType.DMA,
            pltpu.SemaphoreType.DMA,
            pltpu.SemaphoreType.DMA,
            pltpu.SemaphoreType.DMA,
            pltpu.SemaphoreType.DMA,
            pltpu.SemaphoreType.DMA,
            pltpu.SemaphoreType.DMA,
        ],
    )
    def emb_kernel(idx_hbm, table_hbm, out_hbm, idx_v, rows_v, *sems):
        gsem = sems[:NBUF]
        osem = sems[NBUF:]
        cid = lax.axis_index("c")
        sid = lax.axis_index("s")
        wid = sid * info.num_cores + cid
        row_base = wid * b_per_w

        pltpu.sync_copy(idx_hbm.at[pl.ds(wid * g_rows_per_w, g_rows_per_w)],
                        idx_v)

        def fire_gather(ci, s):
            for j in range(K):
                pltpu.async_copy(
                    table_hbm.at[idx_v.at[ci * K + j]],
                    rows_v.at[s, pl.ds(j * G, G)],
                    gsem[s],
                )

        def wait_gather(s):
            pltpu.make_async_copy(
                table_hbm.at[pl.ds(0, C)], rows_v.at[s], gsem[s]).wait()

        def wait_out(ci, s):
            pltpu.make_async_copy(
                rows_v.at[s],
                out_hbm.at[pl.ds(row_base + ci * C, C)],
                osem[s],
            ).wait()

        for b in range(NBUF - 1):
            fire_gather(b, b)

        def group_body(g, carry):
            for b in range(NBUF):
                ci = g * NBUF + b
                wait_gather(b)

                def scale_row(r, carry2):
                    for d in range(D_MODEL // 16):
                        sl = pl.ds(d * 16, 16)
                        rows_v[b, r, sl] = rows_v[b, r, sl] * SCALE
                    return carry2

                lax.fori_loop(0, C, scale_row, 0, unroll=8)

                pltpu.async_copy(
                    rows_v.at[b],
                    out_hbm.at[pl.ds(row_base + ci * C, C)],
                    osem[b],
                )

                s2 = (b + NBUF - 1) % NBUF
                @pl.when(ci >= 1)
                def _():
                    wait_out(ci - 1, s2)

                @pl.when(ci + NBUF - 1 < n_chunks)
                def _():
                    fire_gather(ci + NBUF - 1, s2)
            return carry

        lax.fori_loop(0, n_groups, group_body, 0)
        wait_out(n_chunks - 1, (n_chunks - 1) % NBUF)

    return emb_kernel


def kernel(x, pretrained_vector):
    B = x.shape[0] * x.shape[1]
    idx2d = x.reshape(B // G, G).astype(jnp.int32)
    out = _build(B)(idx2d, pretrained_vector)
    return out.reshape(x.shape[0], x.shape[1], D_MODEL)


# speedup vs baseline: 1.4827x; 1.0036x over previous
import functools
import math

import jax
import jax.numpy as jnp
from jax import lax
from jax.experimental import pallas as pl
from jax.experimental.pallas import tpu as pltpu
from jax.experimental.pallas import tpu_sc as plsc

D_MODEL = 64
SCALE = math.sqrt(D_MODEL)

SEQ = 200
G0 = 104
NBUF = 4


@functools.lru_cache(maxsize=None)
def _build(NB):
    info = plsc.get_sparse_core_info()
    NW = info.num_cores * info.num_subcores
    b_per_w = NB // NW
    n_groups = b_per_w // NBUF

    mesh = plsc.VectorSubcoreMesh(core_axis_name="c", subcore_axis_name="s")

    @functools.partial(
        pl.kernel,
        mesh=mesh,
        compiler_params=pltpu.CompilerParams(use_tc_tiling_on_sc=False),
        out_type=jax.ShapeDtypeStruct((NB, SEQ, D_MODEL), jnp.float32),
        scratch_types=[
            pltpu.VMEM((b_per_w * SEQ,), jnp.int32),
            pltpu.VMEM((NBUF, 1, SEQ, D_MODEL), jnp.float32),
            pltpu.SemaphoreType.DMA,
            pltpu.SemaphoreType.DMA,
            pltpu.SemaphoreType.DMA,
            pltpu.SemaphoreType.DMA,
            pltpu.SemaphoreType.DMA,
            pltpu.SemaphoreType.DMA,
            pltpu.SemaphoreType.DMA,
            pltpu.SemaphoreType.DMA,
        ],
    )
    def emb_kernel(idx_hbm, table_hbm, out_hbm, idx_v, rows_v, *sems):
        gsem = sems[:NBUF]
        osem = sems[NBUF:]
        cid = lax.axis_index("c")
        sid = lax.axis_index("s")
        wid = sid * info.num_cores + cid
        bat_base = wid * b_per_w

        pltpu.sync_copy(
            idx_hbm.at[pl.ds(bat_base * SEQ, b_per_w * SEQ)], idx_v)

        def fire_gather(ci, s):
            for o, g in ((0, G0), (G0, SEQ - G0)):
                pltpu.async_copy(
                    table_hbm.at[idx_v.at[pl.ds(ci * SEQ + o, g)]],
                    rows_v.at[s, 0, pl.ds(o, g)],
                    gsem[s],
                )

        def wait_gather(s):
            pltpu.make_async_copy(
                table_hbm.at[pl.ds(0, SEQ)], rows_v.at[s], gsem[s]).wait()

        def out_slice(ci):
            return out_hbm.at[pl.ds(bat_base + ci, 1)]

        def wait_out(ci, s):
            pltpu.make_async_copy(
                rows_v.at[s], out_slice(ci), osem[s]).wait()

        for b in range(NBUF - 1):
            fire_gather(b, b)

        def group_body(g, carry):
            for b in range(NBUF):
                ci = g * NBUF + b
                wait_gather(b)

                def scale_row(r, carry2):
                    for d in range(D_MODEL // 16):
                        sl = pl.ds(d * 16, 16)
                        rows_v[b, 0, r, sl] = rows_v[b, 0, r, sl] * SCALE
                    return carry2

                lax.fori_loop(0, SEQ, scale_row, 0, unroll=8)

                pltpu.async_copy(rows_v.at[b], out_slice(ci), osem[b])

                s2 = (b + NBUF - 1) % NBUF
                @pl.when(ci >= 1)
                def _():
                    wait_out(ci - 1, s2)

                @pl.when(ci + NBUF - 1 < b_per_w)
                def _():
                    fire_gather(ci + NBUF - 1, s2)
            return carry

        lax.fori_loop(0, n_groups, group_body, 0)
        wait_out(b_per_w - 1, (b_per_w - 1) % NBUF)

    return emb_kernel


def kernel(x, pretrained_vector):
    NB, S = x.shape
    idx = x.reshape(NB * S).astype(jnp.int32)
    return _build(NB)(idx, pretrained_vector)
